# baseline (speedup 1.0000x reference)
import jax
import jax.numpy as jnp
from jax.experimental import pallas as pl
from jax.experimental.pallas import tpu as pltpu

_K = 32
_BT = 64
_BD = 128


def _enc_topk_body(x_ref, we_ref, hp_ref, hs_ref, s16_ref):
    rows = x_ref.shape[0]
    h = jax.lax.dot_general(
        x_ref[...], we_ref[...],
        dimension_numbers=(((1,), (1,)), ((), ())),
        preferred_element_type=jnp.float32,
    )
    hp_ref[...] = h
    cols = hs_ref.shape[1]
    ch = 768
    nch = cols // ch

    def build_hi(j, _):
        sl = pl.dslice(j * ch, ch)
        a = jnp.abs(hp_ref[:, sl])
        hs_ref[:, sl] = a
        b = jax.lax.bitcast_convert_type(a, jnp.int32)
        s16_ref[:, sl] = (b >> 16).astype(jnp.int16)
        return 0

    jax.lax.fori_loop(0, nch, build_hi, 0, unroll=False)

    def step_hi(i, prefix):
        cand = prefix | (jnp.int32(1) << (14 - i))
        cnt = jnp.sum((s16_ref[...] >= cand.astype(jnp.int16)),
                      axis=1, keepdims=True, dtype=jnp.int32)
        return jnp.where(cnt >= _K, cand, prefix)

    p_hi = jax.lax.fori_loop(
        0, 15, step_hi, jnp.zeros((rows, 1), jnp.int32), unroll=False
    )

    c_hi = jnp.sum((s16_ref[...] > p_hi.astype(jnp.int16)),
                   axis=1, keepdims=True, dtype=jnp.int32)

    def build_lo(j, _):
        sl = pl.dslice(j * ch, ch)
        b = jax.lax.bitcast_convert_type(hs_ref[:, sl], jnp.int32)
        lo = jnp.where((b >> 16) == p_hi, (b & 0xFFFF) - 32768, -32768)
        s16_ref[:, sl] = lo.astype(jnp.int16)
        return 0

    jax.lax.fori_loop(0, nch, build_lo, 0, unroll=False)

    def step_lo(i, prefix):
        cand = prefix | (jnp.int32(1) << (15 - i))
        cnt = c_hi + jnp.sum(
            (s16_ref[...] >= (cand - 32768).astype(jnp.int16)),
            axis=1, keepdims=True, dtype=jnp.int32)
        return jnp.where(cnt >= _K, cand, prefix)

    p_lo = jax.lax.fori_loop(
        0, 16, step_lo, jnp.zeros((rows, 1), jnp.int32), unroll=False
    )
    thrf = jax.lax.bitcast_convert_type((p_hi << 16) | p_lo, jnp.float32)

    def finmask(j, _):
        sl = pl.dslice(j * ch, ch)
        hs_ref[:, sl] = jnp.where(hs_ref[:, sl] >= thrf, hp_ref[:, sl], 0.0)
        return 0

    jax.lax.fori_loop(0, nch, finmask, 0, unroll=False)


def _dec_body(hs_ref, wd_ref, b_ref, recon_ref):
    recon_ref[...] = jax.lax.dot_general(
        hs_ref[...], wd_ref[...],
        dimension_numbers=(((1,), (1,)), ((), ())),
        preferred_element_type=jnp.float32,
    ) + b_ref[...]


def kernel(x, W_enc, W_dec, b_dec):
    batch, input_dim = x.shape
    hidden_dim = W_enc.shape[0]
    nb = batch // _BT
    nd = batch // _BD
    b2 = b_dec.reshape(1, input_dim)

    h_pre, h_sparse = pl.pallas_call(
        _enc_topk_body,
        grid=(nb,),
        in_specs=[
            pl.BlockSpec((_BT, input_dim), lambda i: (i, 0)),
            pl.BlockSpec((hidden_dim, input_dim), lambda i: (0, 0)),
        ],
        out_specs=[
            pl.BlockSpec((_BT, hidden_dim), lambda i: (i, 0)),
            pl.BlockSpec((_BT, hidden_dim), lambda i: (i, 0)),
        ],
        out_shape=[
            jax.ShapeDtypeStruct((batch, hidden_dim), jnp.float32),
            jax.ShapeDtypeStruct((batch, hidden_dim), jnp.float32),
        ],
        scratch_shapes=[pltpu.VMEM((_BT, hidden_dim), jnp.int16)],
        compiler_params=pltpu.CompilerParams(
            vmem_limit_bytes=64 * 1024 * 1024,
        ),
    )(x, W_enc)

    recon = pl.pallas_call(
        _dec_body,
        grid=(nd,),
        in_specs=[
            pl.BlockSpec((_BD, hidden_dim), lambda i: (i, 0)),
            pl.BlockSpec((input_dim, hidden_dim), lambda i: (0, 0)),
            pl.BlockSpec((1, input_dim), lambda i: (0, 0)),
        ],
        out_specs=pl.BlockSpec((_BD, input_dim), lambda i: (i, 0)),
        out_shape=jax.ShapeDtypeStruct((batch, input_dim), jnp.float32),
    )(h_sparse, W_dec, b2)

    return (recon, h_sparse, h_pre)


# speedup vs baseline: 1.7918x; 1.7918x over previous
import jax
import jax.numpy as jnp
from jax.experimental import pallas as pl
from jax.experimental.pallas import tpu as pltpu

_K = 32
_BT = 128
_BD = 256


def _enc_topk_body(x_ref, we_ref, hp_ref, hs_ref):
    h = jax.lax.dot_general(
        x_ref[...], we_ref[...],
        dimension_numbers=(((1,), (1,)), ((), ())),
        preferred_element_type=jnp.float32,
    )
    hp_ref[...] = h
    hs_ref[...] = jnp.abs(h)

    def step(i, prefix):
        cand = prefix | (jnp.int32(1) << (30 - i))
        candf = jax.lax.bitcast_convert_type(cand, jnp.float32)
        cnt = jnp.sum((hs_ref[...] >= candf).astype(jnp.int32), axis=1,
                      keepdims=True)
        return jnp.where(cnt >= _K, cand, prefix)

    thr = jax.lax.fori_loop(
        0, 31, step, jnp.zeros((h.shape[0], 1), jnp.int32), unroll=False
    )
    thrf = jax.lax.bitcast_convert_type(thr, jnp.float32)
    hs_ref[...] = jnp.where(hs_ref[...] >= thrf, hp_ref[...], 0.0)


def _dec_body(hs_ref, wd_ref, b_ref, recon_ref):
    recon_ref[...] = jax.lax.dot_general(
        hs_ref[...].astype(jnp.bfloat16), wd_ref[...],
        dimension_numbers=(((1,), (1,)), ((), ())),
        preferred_element_type=jnp.float32,
    ) + b_ref[...]


def kernel(x, W_enc, W_dec, b_dec):
    batch, input_dim = x.shape
    hidden_dim = W_enc.shape[0]
    nb = batch // _BT
    nd = batch // _BD
    b2 = b_dec.reshape(1, input_dim)

    h_pre, h_sparse = pl.pallas_call(
        _enc_topk_body,
        grid=(nb,),
        in_specs=[
            pl.BlockSpec((_BT, input_dim), lambda i: (i, 0)),
            pl.BlockSpec((hidden_dim, input_dim), lambda i: (0, 0)),
        ],
        out_specs=[
            pl.BlockSpec((_BT, hidden_dim), lambda i: (i, 0)),
            pl.BlockSpec((_BT, hidden_dim), lambda i: (i, 0)),
        ],
        out_shape=[
            jax.ShapeDtypeStruct((batch, hidden_dim), jnp.float32),
            jax.ShapeDtypeStruct((batch, hidden_dim), jnp.float32),
        ],
        compiler_params=pltpu.CompilerParams(
            vmem_limit_bytes=64 * 1024 * 1024,
        ),
    )(x, W_enc)

    recon = pl.pallas_call(
        _dec_body,
        grid=(nd,),
        in_specs=[
            pl.BlockSpec((_BD, hidden_dim), lambda i: (i, 0)),
            pl.BlockSpec((input_dim, hidden_dim), lambda i: (0, 0)),
            pl.BlockSpec((1, input_dim), lambda i: (0, 0)),
        ],
        out_specs=pl.BlockSpec((_BD, input_dim), lambda i: (i, 0)),
        out_shape=jax.ShapeDtypeStruct((batch, input_dim), jnp.float32),
        compiler_params=pltpu.CompilerParams(
            vmem_limit_bytes=64 * 1024 * 1024,
        ),
    )(h_sparse, W_dec.astype(jnp.bfloat16), b2)

    return (recon, h_sparse, h_pre)
